# baseline (speedup 1.0000x reference)
import functools

import jax
import jax.numpy as jnp
from jax import lax
from jax.experimental import pallas as pl
from jax.experimental.pallas import tpu as pltpu
from jax.experimental.pallas import tpu_sc as plsc

B = 100000
D = 256
V = 100
C = 160
CG = 0
NC = 2
NS = 16
NW = NC * NS
L = 16
NUM_CHUNKS = B // C
T = -(-NUM_CHUNKS // NW)
NBUF = 2


@functools.partial(
    pl.kernel,
    mesh=plsc.VectorSubcoreMesh(core_axis_name="c", subcore_axis_name="s"),
    out_type=jax.ShapeDtypeStruct((B, D), jnp.float32),
    compiler_params=pltpu.CompilerParams(needs_layout_passes=False),
    scratch_types=(
        [pltpu.VMEM((V, D), jnp.float32)]
        + [pltpu.VMEM((C,), jnp.int32)] * NBUF
        + [pltpu.VMEM((C, D), jnp.float32)] * NBUF
        + [pltpu.SemaphoreType.DMA] * (3 * NBUF)
    ),
)
def _embed_kernel(idx_hbm, table_hbm, out_hbm, *scratch):
    table_v = scratch[0]
    idx_v = scratch[1:1 + NBUF]
    rows_v = scratch[1 + NBUF:1 + 2 * NBUF]
    isem = scratch[1 + 2 * NBUF:1 + 3 * NBUF]
    gsem = scratch[1 + 3 * NBUF:1 + 4 * NBUF]
    osem = scratch[1 + 4 * NBUF:1 + 5 * NBUF]

    wid = lax.axis_index("s") * NC + lax.axis_index("c")

    def start_idx(b, cid):
        pltpu.async_copy(idx_hbm.at[pl.ds(cid * C, C)], idx_v[b], isem[b])

    def wait_idx(b):
        pltpu.make_async_copy(idx_hbm.at[pl.ds(0, C)],
                              idx_v[b], isem[b]).wait()

    def start_gather(b):
        pltpu.async_copy(table_hbm.at[idx_v[b].at[pl.ds(0, CG)]],
                         rows_v[b].at[pl.ds(0, CG)], gsem[b])

    def wait_gather(b):
        pltpu.make_async_copy(out_hbm.at[pl.ds(0, CG)],
                              rows_v[b].at[pl.ds(0, CG)], gsem[b]).wait()

    def start_store(b, cid):
        pltpu.async_copy(rows_v[b], out_hbm.at[pl.ds(cid * C, C)], osem[b])

    def wait_store(b):
        pltpu.make_async_copy(rows_v[b],
                              out_hbm.at[pl.ds(0, C)], osem[b]).wait()

    def expand(b):
        ib = idx_v[b]
        rb = rows_v[b]

        @plsc.parallel_loop(CG // L, C // L, unroll=2)
        def _group(g):
            ivec = ib[pl.ds(g * L, L)]
            for l in range(L):
                tok = ivec[l]
                r = g * L + l
                vs = [table_v[tok, pl.ds(L * j, L)] for j in range(D // L)]
                for j in range(D // L):
                    rb[r, pl.ds(L * j, L)] = vs[j]

    start_idx(0, wid)
    start_idx(1, wid + NW)
    pltpu.sync_copy(table_hbm, table_v)

    @pl.loop(0, T, step=NBUF)
    def _pair(t0):
        for b in range(NBUF):
            t = t0 + b
            cid = wid + t * NW

            @pl.when(cid < NUM_CHUNKS)
            def _chunk(t=t, cid=cid, b=b):
                wait_idx(b)

                @pl.when(t >= NBUF)
                def _free_rows():
                    wait_store(b)

                if CG:
                    start_gather(b)
                expand(b)
                if CG:
                    wait_gather(b)
                start_store(b, cid)

                @pl.when(cid + NBUF * NW < NUM_CHUNKS)
                def _prefetch():
                    start_idx(b, cid + NBUF * NW)

    wait_store(0)
    wait_store(1)


def kernel(atomic_numbers, table):
    idx = atomic_numbers.astype(jnp.int32)
    return _embed_kernel(idx, table)


# speedup vs baseline: 1.0922x; 1.0922x over previous
import functools

import jax
import jax.numpy as jnp
from jax import lax
from jax.experimental import pallas as pl
from jax.experimental.pallas import tpu as pltpu
from jax.experimental.pallas import tpu_sc as plsc

B = 100000
D = 256
V = 100
C = 160
CG = 0
NC = 2
NS = 16
NW = NC * NS
L = 16
NUM_CHUNKS = B // C
T = -(-NUM_CHUNKS // NW)
NBUF = 2


@functools.partial(
    pl.kernel,
    mesh=plsc.VectorSubcoreMesh(core_axis_name="c", subcore_axis_name="s"),
    out_type=jax.ShapeDtypeStruct((B, D), jnp.float32),
    compiler_params=pltpu.CompilerParams(needs_layout_passes=False),
    scratch_types=(
        [pltpu.VMEM((V, D), jnp.float32)]
        + [pltpu.VMEM((C,), jnp.int32)] * NBUF
        + [pltpu.VMEM((C, D), jnp.float32)] * NBUF
        + [pltpu.SemaphoreType.DMA] * (3 * NBUF)
    ),
)
def _embed_kernel(idx_hbm, table_hbm, out_hbm, *scratch):
    table_v = scratch[0]
    idx_v = scratch[1:1 + NBUF]
    rows_v = scratch[1 + NBUF:1 + 2 * NBUF]
    isem = scratch[1 + 2 * NBUF:1 + 3 * NBUF]
    gsem = scratch[1 + 3 * NBUF:1 + 4 * NBUF]
    osem = scratch[1 + 4 * NBUF:1 + 5 * NBUF]

    wid = lax.axis_index("s") * NC + lax.axis_index("c")

    def start_idx(b, cid):
        pltpu.async_copy(idx_hbm.at[pl.ds(cid * C, C)], idx_v[b], isem[b])

    def wait_idx(b):
        pltpu.make_async_copy(idx_hbm.at[pl.ds(0, C)],
                              idx_v[b], isem[b]).wait()

    def start_gather(b):
        pltpu.async_copy(table_hbm.at[idx_v[b].at[pl.ds(0, CG)]],
                         rows_v[b].at[pl.ds(0, CG)], gsem[b])

    def wait_gather(b):
        pltpu.make_async_copy(out_hbm.at[pl.ds(0, CG)],
                              rows_v[b].at[pl.ds(0, CG)], gsem[b]).wait()

    def start_store(b, cid):
        pltpu.async_copy(rows_v[b], out_hbm.at[pl.ds(cid * C, C)], osem[b])

    def wait_store(b):
        pltpu.make_async_copy(rows_v[b],
                              out_hbm.at[pl.ds(0, C)], osem[b]).wait()

    def expand(b):
        ib = idx_v[b]
        rb = rows_v[b]

        @plsc.parallel_loop(CG // L, C // L)
        def _group(g):
            ivec = ib[pl.ds(g * L, L)]
            for l in range(0, L, 2):
                tok0 = ivec[l]
                tok1 = ivec[l + 1]
                r = g * L + l
                v0 = [table_v[tok0, pl.ds(L * j, L)] for j in range(D // L)]
                v1 = [table_v[tok1, pl.ds(L * j, L)] for j in range(D // L)]
                for j in range(D // L):
                    rb[r, pl.ds(L * j, L)] = v0[j]
                for j in range(D // L):
                    rb[r + 1, pl.ds(L * j, L)] = v1[j]

    start_idx(0, wid)
    start_idx(1, wid + NW)
    pltpu.sync_copy(table_hbm, table_v)

    @pl.loop(0, T, step=NBUF)
    def _pair(t0):
        for b in range(NBUF):
            t = t0 + b
            cid = wid + t * NW

            @pl.when(cid < NUM_CHUNKS)
            def _chunk(t=t, cid=cid, b=b):
                wait_idx(b)

                @pl.when(t >= NBUF)
                def _free_rows():
                    wait_store(b)

                if CG:
                    start_gather(b)
                expand(b)
                if CG:
                    wait_gather(b)
                start_store(b, cid)

                @pl.when(cid + NBUF * NW < NUM_CHUNKS)
                def _prefetch():
                    start_idx(b, cid + NBUF * NW)

    wait_store(0)
    wait_store(1)


def kernel(atomic_numbers, table):
    idx = atomic_numbers.astype(jnp.int32)
    return _embed_kernel(idx, table)


# speedup vs baseline: 1.1003x; 1.0074x over previous
import functools

import jax
import jax.numpy as jnp
from jax import lax
from jax.experimental import pallas as pl
from jax.experimental.pallas import tpu as pltpu
from jax.experimental.pallas import tpu_sc as plsc

B = 100000
D = 256
V = 100
C = 160
NC = 2
NS = 16
NW = NC * NS
L = 16
NUM_CHUNKS = B // C
T = -(-NUM_CHUNKS // NW)
NBUF = 2


@functools.partial(
    pl.kernel,
    mesh=plsc.VectorSubcoreMesh(core_axis_name="c", subcore_axis_name="s"),
    out_type=jax.ShapeDtypeStruct((B, D), jnp.float32),
    compiler_params=pltpu.CompilerParams(needs_layout_passes=False),
    scratch_types=(
        [pltpu.VMEM((V, D), jnp.float32)]
        + [pltpu.VMEM((C,), jnp.int32)] * NBUF
        + [pltpu.VMEM((C, D), jnp.float32)] * NBUF
        + [pltpu.SemaphoreType.DMA] * (2 * NBUF)
    ),
)
def _embed_kernel(idx_hbm, table_hbm, out_hbm, *scratch):
    table_v = scratch[0]
    idx_v = scratch[1:1 + NBUF]
    rows_v = scratch[1 + NBUF:1 + 2 * NBUF]
    isem = scratch[1 + 2 * NBUF:1 + 3 * NBUF]
    osem = scratch[1 + 3 * NBUF:1 + 4 * NBUF]

    wid = lax.axis_index("s") * NC + lax.axis_index("c")

    def start_idx(b, cid):
        pltpu.async_copy(idx_hbm.at[pl.ds(cid * C, C)], idx_v[b], isem[b])

    def wait_idx(b):
        pltpu.make_async_copy(idx_hbm.at[pl.ds(0, C)],
                              idx_v[b], isem[b]).wait()

    def start_store(b, cid):
        pltpu.async_copy(rows_v[b], out_hbm.at[pl.ds(cid * C, C)], osem[b])

    def wait_store(b):
        pltpu.make_async_copy(rows_v[b],
                              out_hbm.at[pl.ds(0, C)], osem[b]).wait()

    def expand(b):
        ib = idx_v[b]
        rb = rows_v[b]

        @plsc.parallel_loop(0, C // L)
        def _group(g):
            ivec = ib[pl.ds(g * L, L)]
            for l in range(L):
                tok = ivec[l]
                r = g * L + l
                vs = [table_v[tok, pl.ds(L * j, L)] for j in range(D // L)]
                for j in range(D // L):
                    rb[r, pl.ds(L * j, L)] = vs[j]

    start_idx(0, wid)
    start_idx(1, wid + NW)
    pltpu.sync_copy(table_hbm, table_v)

    @pl.loop(0, T, step=NBUF)
    def _pair(t0):
        for b in range(NBUF):
            t = t0 + b
            cid = wid + t * NW

            @pl.when(cid < NUM_CHUNKS)
            def _chunk(t=t, cid=cid, b=b):
                wait_idx(b)

                @pl.when(t >= NBUF)
                def _free_rows():
                    wait_store(b)

                expand(b)
                start_store(b, cid)

                @pl.when(cid + NBUF * NW < NUM_CHUNKS)
                def _prefetch():
                    start_idx(b, cid + NBUF * NW)

    wait_store(0)
    wait_store(1)


def kernel(atomic_numbers, table):
    idx = atomic_numbers.astype(jnp.int32)
    return _embed_kernel(idx, table)


# speedup vs baseline: 1.1291x; 1.0262x over previous
import functools

import jax
import jax.numpy as jnp
from jax import lax
from jax.experimental import pallas as pl
from jax.experimental.pallas import tpu as pltpu
from jax.experimental.pallas import tpu_sc as plsc

B = 100000
D = 256
V = 100
C = 160
NC = 2
NS = 16
NW = NC * NS
L = 16
NUM_CHUNKS = B // C
T = -(-NUM_CHUNKS // NW)
NBUF = 2


@functools.partial(
    pl.kernel,
    mesh=plsc.VectorSubcoreMesh(core_axis_name="c", subcore_axis_name="s"),
    out_type=jax.ShapeDtypeStruct((B, D), jnp.float32),
    compiler_params=pltpu.CompilerParams(needs_layout_passes=False),
    scratch_types=(
        [pltpu.VMEM((V, D), jnp.float32)]
        + [pltpu.VMEM((C,), jnp.int32)] * NBUF
        + [pltpu.VMEM((C, D), jnp.float32)] * NBUF
        + [pltpu.SemaphoreType.DMA] * (2 * NBUF)
    ),
)
def _embed_kernel(idx_hbm, table_hbm, out_hbm, *scratch):
    table_v = scratch[0]
    idx_v = scratch[1:1 + NBUF]
    rows_v = scratch[1 + NBUF:1 + 2 * NBUF]
    isem = scratch[1 + 2 * NBUF:1 + 3 * NBUF]
    osem = scratch[1 + 3 * NBUF:1 + 4 * NBUF]

    wid = lax.axis_index("s") * NC + lax.axis_index("c")

    def start_idx(b, cid):
        pltpu.async_copy(idx_hbm.at[pl.ds(cid * C, C)], idx_v[b], isem[b])

    def wait_idx(b):
        pltpu.make_async_copy(idx_hbm.at[pl.ds(0, C)],
                              idx_v[b], isem[b]).wait()

    def start_store(b, cid):
        pltpu.async_copy(rows_v[b], out_hbm.at[pl.ds(cid * C, C)], osem[b])

    def wait_store(b):
        pltpu.make_async_copy(rows_v[b],
                              out_hbm.at[pl.ds(0, C)], osem[b]).wait()

    def expand(b):
        ib = idx_v[b]
        rb = rows_v[b]

        @plsc.parallel_loop(0, C // L)
        def _group(g):
            ivec = ib[pl.ds(g * L, L)]
            for l in range(L):
                tok = ivec[l]
                r = g * L + l
                for h in range(0, D // L, 8):
                    vs = [table_v[tok, pl.ds(L * j, L)]
                          for j in range(h, h + 8)]
                    for j in range(h, h + 8):
                        rb[r, pl.ds(L * j, L)] = vs[j - h]

    start_idx(0, wid)
    start_idx(1, wid + NW)
    pltpu.sync_copy(table_hbm, table_v)

    @pl.loop(0, T, step=NBUF)
    def _pair(t0):
        for b in range(NBUF):
            t = t0 + b
            cid = wid + t * NW

            @pl.when(cid < NUM_CHUNKS)
            def _chunk(t=t, cid=cid, b=b):
                wait_idx(b)

                @pl.when(t >= NBUF)
                def _free_rows():
                    wait_store(b)

                expand(b)
                start_store(b, cid)

                @pl.when(cid + NBUF * NW < NUM_CHUNKS)
                def _prefetch():
                    start_idx(b, cid + NBUF * NW)

    wait_store(0)
    wait_store(1)


def kernel(atomic_numbers, table):
    idx = atomic_numbers.astype(jnp.int32)
    return _embed_kernel(idx, table)


# speedup vs baseline: 1.2277x; 1.0873x over previous
import functools

import jax
import jax.numpy as jnp
from jax import lax
from jax.experimental import pallas as pl
from jax.experimental.pallas import tpu as pltpu
from jax.experimental.pallas import tpu_sc as plsc

B = 100000
D = 256
V = 100
C = 160
NC = 2
NS = 16
NW = NC * NS
L = 16
NUM_CHUNKS = B // C
T = -(-NUM_CHUNKS // NW)
NBUF = 2


@functools.partial(
    pl.kernel,
    mesh=plsc.VectorSubcoreMesh(core_axis_name="c", subcore_axis_name="s"),
    out_type=jax.ShapeDtypeStruct((B, D), jnp.float32),
    compiler_params=pltpu.CompilerParams(needs_layout_passes=False),
    scratch_types=(
        [pltpu.VMEM((V, D), jnp.float32)]
        + [pltpu.VMEM((C,), jnp.int32)] * NBUF
        + [pltpu.VMEM((C, D), jnp.float32)] * NBUF
        + [pltpu.SemaphoreType.DMA] * (2 * NBUF)
    ),
)
def _embed_kernel(idx_hbm, table_hbm, out_hbm, *scratch):
    table_v = scratch[0]
    idx_v = scratch[1:1 + NBUF]
    rows_v = scratch[1 + NBUF:1 + 2 * NBUF]
    isem = scratch[1 + 2 * NBUF:1 + 3 * NBUF]
    osem = scratch[1 + 3 * NBUF:1 + 4 * NBUF]

    wid = lax.axis_index("s") * NC + lax.axis_index("c")

    def start_idx(b, cid):
        pltpu.async_copy(idx_hbm.at[pl.ds(cid * C, C)], idx_v[b], isem[b])

    def wait_idx(b):
        pltpu.make_async_copy(idx_hbm.at[pl.ds(0, C)],
                              idx_v[b], isem[b]).wait()

    def start_store(b, cid):
        pltpu.async_copy(rows_v[b], out_hbm.at[pl.ds(cid * C, C)], osem[b])

    def wait_store(b):
        pltpu.make_async_copy(rows_v[b],
                              out_hbm.at[pl.ds(0, C)], osem[b]).wait()

    def expand(b):
        ib = idx_v[b]
        rb = rows_v[b]

        @plsc.parallel_loop(0, C // L)
        def _group(g):
            ivec = ib[pl.ds(g * L, L)]
            for l in range(L):
                tok = ivec[l]
                r = g * L + l
                for h in range(0, D // L, 4):
                    vs = [table_v[tok, pl.ds(L * j, L)]
                          for j in range(h, h + 4)]
                    for j in range(h, h + 4):
                        rb[r, pl.ds(L * j, L)] = vs[j - h]

    start_idx(0, wid)
    start_idx(1, wid + NW)
    pltpu.sync_copy(table_hbm, table_v)

    @pl.loop(0, T, step=NBUF)
    def _pair(t0):
        for b in range(NBUF):
            t = t0 + b
            cid = wid + t * NW

            @pl.when(cid < NUM_CHUNKS)
            def _chunk(t=t, cid=cid, b=b):
                wait_idx(b)

                @pl.when(t >= NBUF)
                def _free_rows():
                    wait_store(b)

                expand(b)
                start_store(b, cid)

                @pl.when(cid + NBUF * NW < NUM_CHUNKS)
                def _prefetch():
                    start_idx(b, cid + NBUF * NW)

    wait_store(0)
    wait_store(1)


def kernel(atomic_numbers, table):
    idx = atomic_numbers.astype(jnp.int32)
    return _embed_kernel(idx, table)
